# baseline (speedup 1.0000x reference)
import functools

import jax
import jax.numpy as jnp
from jax import lax
from jax.experimental import pallas as pl
from jax.experimental.pallas import tpu as pltpu
from jax.experimental.pallas import tpu_sc as plsc

N_NODES = 512
KSIZE = 16
N_CHAN = 8
N_STEPS = 2
NUM_WORKERS = 32
ROWS_PER_W = N_NODES // NUM_WORKERS


def _worker_base():
    wid = lax.axis_index("s") * 2 + lax.axis_index("c")
    return wid * ROWS_PER_W


def _masked_rowsum(m_v, x_v, r):
    def body(t, acc):
        mv = m_v[r, pl.ds(t * 16, 16)]
        mf = jnp.minimum(mv, 1).astype(jnp.float32)
        terms = [x_v[t * 16 + l, :] * mf[l] for l in range(16)]
        while len(terms) > 1:
            terms = [terms[i] + terms[i + 1] for i in range(0, len(terms), 2)]
        return acc + terms[0]

    return lax.fori_loop(0, N_NODES // 16, body,
                         jnp.zeros((KSIZE,), jnp.float32))


@functools.cache
def _build_calls():
    mesh = plsc.VectorSubcoreMesh(core_axis_name="c", subcore_axis_name="s")

    @functools.partial(
        pl.kernel,
        out_type=jax.ShapeDtypeStruct((N_NODES, KSIZE), jnp.float32),
        mesh=mesh,
        scratch_types=[
            pltpu.VMEM((ROWS_PER_W, N_NODES), jnp.int32),
            pltpu.VMEM((N_NODES, KSIZE), jnp.float32),
            pltpu.VMEM((ROWS_PER_W, KSIZE), jnp.float32),
        ],
    )
    def aggregate(m_hbm, x_hbm, out_hbm, m_v, x_v, o_v):
        base = _worker_base()
        pltpu.sync_copy(m_hbm.at[pl.ds(base, ROWS_PER_W), :], m_v)
        pltpu.sync_copy(x_hbm, x_v)
        for r in range(ROWS_PER_W):
            o_v[r, :] = _masked_rowsum(m_v, x_v, r)
        pltpu.sync_copy(o_v, out_hbm.at[pl.ds(base, ROWS_PER_W), :])

    @functools.partial(
        pl.kernel,
        out_type=jax.ShapeDtypeStruct((N_CHAN * N_NODES, KSIZE), jnp.float32),
        mesh=mesh,
        scratch_types=[
            pltpu.VMEM((ROWS_PER_W, N_NODES), jnp.int32),
            pltpu.VMEM((N_NODES, KSIZE), jnp.float32),
            pltpu.VMEM((ROWS_PER_W, KSIZE), jnp.float32),
            pltpu.VMEM((N_CHAN * N_STEPS, KSIZE), jnp.float32),
            pltpu.VMEM((N_CHAN, ROWS_PER_W, KSIZE), jnp.float32),
        ],
    )
    def aggregate_combine(m_hbm, p_hbm, l_hbm, k_hbm, out_hbm,
                          m_v, p_v, l_v, k_v, o_v):
        base = _worker_base()
        pltpu.sync_copy(m_hbm.at[pl.ds(base, ROWS_PER_W), :], m_v)
        pltpu.sync_copy(p_hbm, p_v)
        pltpu.sync_copy(l_hbm.at[pl.ds(base, ROWS_PER_W), :], l_v)
        pltpu.sync_copy(k_hbm, k_v)
        for r in range(ROWS_PER_W):
            q = _masked_rowsum(m_v, p_v, r)
            p_i = p_v[base + r, :]
            l_i = l_v[r, :]
            for c in range(N_CHAN):
                k0 = k_v[2 * c, :]
                k1 = k_v[2 * c + 1, :]
                o_v[c, r, :] = l_i + p_i * (k0 + k1) + q * (k0 * k1)
        for c in range(N_CHAN):
            pltpu.sync_copy(
                o_v.at[c],
                out_hbm.at[pl.ds(c * N_NODES + base, ROWS_PER_W), :])

    return aggregate, aggregate_combine


def kernel(labelsList, ligand_structure, kernels):
    aggregate, aggregate_combine = _build_calls()
    p = aggregate(ligand_structure, labelsList)
    flat_k = kernels.reshape(N_CHAN * N_STEPS, KSIZE)
    out = aggregate_combine(ligand_structure, p, labelsList, flat_k)
    return out.reshape(N_CHAN, N_NODES, KSIZE)
